# baseline (speedup 1.0000x reference)
import dataclasses
import functools

import jax
import jax.numpy as jnp
from jax import lax
from jax.experimental import pallas as pl
from jax.experimental.pallas import tpu as pltpu
from jax.experimental.pallas import tpu_sc as plsc

_N = 100000
_D = 128
_DEG = 16
_NB = 4096
_BS = 32

_W = 32
_ABLK = 20480
_NPAD = 5 * _ABLK
_NPW = _NPAD // _W
_CHB = 640
_BPW = _NB // _W
_L = 16
_TSTART = (_N // 128) * 128
_TW = 128


def _compiler_params():
    cp = pltpu.CompilerParams()
    if "needs_layout_passes" in pltpu.CompilerParams.__dataclass_fields__:
        cp = dataclasses.replace(cp, needs_layout_passes=False)
    return cp


def _proj_body(xT_ref, t_ref, h_ref):
    h_ref[...] = jnp.sum(xT_ref[...] * t_ref[...], axis=0)


_proj = pl.pallas_call(
    _proj_body,
    grid=(_NPAD // _ABLK,),
    in_specs=[
        pl.BlockSpec((_D, _ABLK), lambda i: (0, i)),
        pl.BlockSpec((_D, 1), lambda i: (0, 0)),
    ],
    out_specs=pl.BlockSpec((_ABLK,), lambda i: (i,)),
    out_shape=jax.ShapeDtypeStruct((_NPAD,), jnp.float32),
    compiler_params=pltpu.CompilerParams(
        dimension_semantics=("parallel",)),
)


def _make_ns_kernel():
    mesh = plsc.VectorSubcoreMesh(core_axis_name="c", subcore_axis_name="s")

    @functools.partial(
        pl.kernel,
        out_type=jax.ShapeDtypeStruct((_NPAD,), jnp.float32),
        mesh=mesh,
        compiler_params=_compiler_params(),
        scratch_types=[
            pltpu.VMEM((_N,), jnp.float32),
            pltpu.VMEM((_DEG, _CHB), jnp.int32),
            pltpu.VMEM((_DEG, _CHB), jnp.int32),
            pltpu.VMEM((_CHB,), jnp.float32),
            pltpu.VMEM((_DEG, _TW), jnp.int32),
            pltpu.VMEM((_TW,), jnp.float32),
            pltpu.SemaphoreType.DMA,
            pltpu.SemaphoreType.DMA,
            pltpu.SemaphoreType.DMA,
        ],
    )
    def ns_kernel(nbrT_hbm, tail_hbm, nw_hbm, out_hbm, nw_v, nbr0_v, nbr1_v,
                  o_v, t_v, to_v, sem_nw, sem0, sem1):
        wid = lax.axis_index("s") * 2 + lax.axis_index("c")
        nw_cp = pltpu.async_copy(nw_hbm, nw_v, sem_nw)
        base0 = wid * _NPW
        nchunks = _NPW // _CHB
        bufs = (nbr0_v, nbr1_v)
        sems = (sem0, sem1)

        def chunk_base(c):
            return jnp.minimum(base0 + c * _CHB, _TSTART - _CHB)

        cps = [None, None]
        cps[0] = pltpu.async_copy(
            nbrT_hbm.at[:, pl.ds(chunk_base(0), _CHB)], bufs[0], sems[0])
        nw_cp.wait()
        for c in range(nchunks):
            if c + 1 < nchunks:
                nxt = (c + 1) % 2
                cps[nxt] = pltpu.async_copy(
                    nbrT_hbm.at[:, pl.ds(chunk_base(c + 1), _CHB)],
                    bufs[nxt], sems[nxt])
            cps[c % 2].wait()
            nbr_v = bufs[c % 2]

            @pl.loop(0, _CHB // _L)
            def _(i):
                o = i * _L
                accs = [plsc.load_gather(nw_v, [nbr_v[d, pl.ds(o, _L)]])
                        for d in range(4)]
                for d in range(4, _DEG):
                    accs[d % 4] = accs[d % 4] + plsc.load_gather(
                        nw_v, [nbr_v[d, pl.ds(o, _L)]])
                o_v[pl.ds(o, _L)] = ((accs[0] + accs[1])
                                     + (accs[2] + accs[3]))

            pltpu.sync_copy(o_v, out_hbm.at[pl.ds(chunk_base(c), _CHB)])

        @pl.when(wid == _W - 1)
        def _():
            pltpu.sync_copy(tail_hbm, t_v)

            @pl.loop(0, _TW // _L)
            def _(i):
                o = i * _L
                acc = plsc.load_gather(nw_v, [t_v[0, pl.ds(o, _L)]])
                for d in range(1, _DEG):
                    acc = acc + plsc.load_gather(nw_v,
                                                 [t_v[d, pl.ds(o, _L)]])
                to_v[pl.ds(o, _L)] = acc

            pltpu.sync_copy(to_v, out_hbm.at[pl.ds(_TSTART, _TW)])

    return ns_kernel


_ns_cache = functools.cache(_make_ns_kernel)


def _make_mul_kernel():
    mesh = plsc.VectorSubcoreMesh(core_axis_name="c", subcore_axis_name="s")

    @functools.partial(
        pl.kernel,
        out_type=jax.ShapeDtypeStruct((_NPAD,), jnp.float32),
        mesh=mesh,
        compiler_params=_compiler_params(),
        scratch_types=[
            pltpu.VMEM((_NPW,), jnp.float32),
            pltpu.VMEM((_NPW,), jnp.float32),
        ],
    )
    def mul_kernel(h_hbm, ns_hbm, out_hbm, h_v, ns_v):
        wid = lax.axis_index("s") * 2 + lax.axis_index("c")
        base = wid * _NPW
        pltpu.sync_copy(h_hbm.at[pl.ds(base, _NPW)], h_v)
        pltpu.sync_copy(ns_hbm.at[pl.ds(base, _NPW)], ns_v)

        @pl.loop(0, _NPW // _L)
        def _(i):
            o = i * _L
            ns_v[pl.ds(o, _L)] = ns_v[pl.ds(o, _L)] * h_v[pl.ds(o, _L)]

        pltpu.sync_copy(ns_v, out_hbm.at[pl.ds(base, _NPW)])

    return mul_kernel


_mul_cache = functools.cache(_make_mul_kernel)


def _make_score_kernel():
    mesh = plsc.VectorSubcoreMesh(core_axis_name="c", subcore_axis_name="s")

    @functools.partial(
        pl.kernel,
        out_type=jax.ShapeDtypeStruct((_NB,), jnp.float32),
        mesh=mesh,
        compiler_params=_compiler_params(),
        scratch_types=[
            pltpu.VMEM((_NPAD,), jnp.float32),
            pltpu.VMEM((_BS, _BPW), jnp.int32),
            pltpu.VMEM((_BS, _BPW), jnp.float32),
            pltpu.VMEM((_BPW,), jnp.float32),
            pltpu.SemaphoreType.DMA,
        ],
    )
    def score_kernel(comb_hbm, bagsT_hbm, alphaT_hbm, out_hbm, tab_v, idx_v,
                     a_v, o_v, sem):
        wid = lax.axis_index("s") * 2 + lax.axis_index("c")
        base = wid * _BPW
        pltpu.sync_copy(bagsT_hbm.at[:, pl.ds(base, _BPW)], idx_v)
        pltpu.sync_copy(alphaT_hbm.at[:, pl.ds(base, _BPW)], a_v)
        pltpu.async_copy(comb_hbm, tab_v, sem).wait()

        @pl.loop(0, _BPW // _L)
        def _(i):
            o = i * _L
            acc = (plsc.load_gather(tab_v, [idx_v[0, pl.ds(o, _L)]])
                   * a_v[0, pl.ds(o, _L)])
            for j in range(1, _BS):
                acc = acc + (plsc.load_gather(tab_v, [idx_v[j, pl.ds(o, _L)]])
                             * a_v[j, pl.ds(o, _L)])
            o_v[pl.ds(o, _L)] = acc

        pltpu.sync_copy(o_v, out_hbm.at[pl.ds(base, _BPW)])

    return score_kernel


_score_cache = functools.cache(_make_score_kernel)


def kernel(x, sampled_bags, alpha_values, theta_w, node_weights, neighbors):
    h = _proj(x.T, theta_w)
    nbrT = neighbors.T
    tail = jnp.pad(neighbors[_TSTART:].T,
                   ((0, 0), (0, _TW - (_N - _TSTART))))
    ns = _ns_cache()(nbrT, tail, node_weights)
    comb = _mul_cache()(h, ns)
    bagsT = sampled_bags.T
    alphaT = alpha_values[:, :, 0].T
    return _score_cache()(comb, bagsT, alphaT)


# speedup vs baseline: 1.5897x; 1.5897x over previous
import dataclasses
import functools

import jax
import jax.numpy as jnp
from jax import lax
from jax.experimental import pallas as pl
from jax.experimental.pallas import tpu as pltpu
from jax.experimental.pallas import tpu_sc as plsc

_N = 100000
_D = 128
_DEG = 16
_NB = 4096
_BS = 32

_W = 32
_ABLK = 20480
_NPAD = 5 * _ABLK
_NPW = _NPAD // _W
_CHB = 640
_BPW = _NB // _W
_L = 16
_TSTART = (_N // 128) * 128
_TW = 128


def _compiler_params():
    cp = pltpu.CompilerParams()
    if "needs_layout_passes" in pltpu.CompilerParams.__dataclass_fields__:
        cp = dataclasses.replace(cp, needs_layout_passes=False)
    return cp


def _proj_body(x_ref, t_ref, h_ref):
    h = lax.dot_general(
        t_ref[...], x_ref[...], (((0,), (1,)), ((), ())),
        preferred_element_type=jnp.float32)
    h_ref[...] = h[0, :]


_proj = pl.pallas_call(
    _proj_body,
    grid=(_NPAD // _ABLK,),
    in_specs=[
        pl.BlockSpec((_ABLK, _D), lambda i: (i, 0)),
        pl.BlockSpec((_D, 1), lambda i: (0, 0)),
    ],
    out_specs=pl.BlockSpec((_ABLK,), lambda i: (i,)),
    out_shape=jax.ShapeDtypeStruct((_NPAD,), jnp.float32),
    compiler_params=pltpu.CompilerParams(
        dimension_semantics=("parallel",)),
)


def _make_ns_kernel():
    mesh = plsc.VectorSubcoreMesh(core_axis_name="c", subcore_axis_name="s")

    @functools.partial(
        pl.kernel,
        out_type=jax.ShapeDtypeStruct((_NPAD,), jnp.float32),
        mesh=mesh,
        compiler_params=_compiler_params(),
        scratch_types=[
            pltpu.VMEM((_N,), jnp.float32),
            pltpu.VMEM((_DEG, _CHB), jnp.int32),
            pltpu.VMEM((_DEG, _CHB), jnp.int32),
            pltpu.VMEM((_CHB,), jnp.float32),
            pltpu.VMEM((_DEG, _TW), jnp.int32),
            pltpu.VMEM((_TW,), jnp.float32),
            pltpu.SemaphoreType.DMA,
            pltpu.SemaphoreType.DMA,
            pltpu.SemaphoreType.DMA,
        ],
    )
    def ns_kernel(nbrT_hbm, tail_hbm, nw_hbm, out_hbm, nw_v, nbr0_v, nbr1_v,
                  o_v, t_v, to_v, sem_nw, sem0, sem1):
        wid = lax.axis_index("s") * 2 + lax.axis_index("c")
        nw_cps = []
        for st, ln in ((0, 25088), (25088, 25088), (50176, 25088),
                       (75264, _N - 75264)):
            nw_cps.append(pltpu.async_copy(
                nw_hbm.at[pl.ds(st, ln)], nw_v.at[pl.ds(st, ln)], sem_nw))
        base0 = wid * _NPW
        nchunks = _NPW // _CHB
        bufs = (nbr0_v, nbr1_v)
        sems = (sem0, sem1)

        def chunk_base(c):
            return jnp.minimum(base0 + c * _CHB, _TSTART - _CHB)

        cps = [None, None]
        cps[0] = pltpu.async_copy(
            nbrT_hbm.at[:, pl.ds(chunk_base(0), _CHB)], bufs[0], sems[0])
        for cp in nw_cps:
            cp.wait()
        for c in range(nchunks):
            if c + 1 < nchunks:
                nxt = (c + 1) % 2
                cps[nxt] = pltpu.async_copy(
                    nbrT_hbm.at[:, pl.ds(chunk_base(c + 1), _CHB)],
                    bufs[nxt], sems[nxt])
            cps[c % 2].wait()
            nbr_v = bufs[c % 2]

            @pl.loop(0, _CHB // _L)
            def _(i):
                o = i * _L
                accs = [plsc.load_gather(nw_v, [nbr_v[d, pl.ds(o, _L)]])
                        for d in range(4)]
                for d in range(4, _DEG):
                    accs[d % 4] = accs[d % 4] + plsc.load_gather(
                        nw_v, [nbr_v[d, pl.ds(o, _L)]])
                o_v[pl.ds(o, _L)] = ((accs[0] + accs[1])
                                     + (accs[2] + accs[3]))

            pltpu.sync_copy(o_v, out_hbm.at[pl.ds(chunk_base(c), _CHB)])

        @pl.when(wid == _W - 1)
        def _():
            pltpu.sync_copy(tail_hbm, t_v)

            @pl.loop(0, _TW // _L)
            def _(i):
                o = i * _L
                acc = plsc.load_gather(nw_v, [t_v[0, pl.ds(o, _L)]])
                for d in range(1, _DEG):
                    acc = acc + plsc.load_gather(nw_v,
                                                 [t_v[d, pl.ds(o, _L)]])
                to_v[pl.ds(o, _L)] = acc

            pltpu.sync_copy(to_v, out_hbm.at[pl.ds(_TSTART, _TW)])

    return ns_kernel


_ns_cache = functools.cache(_make_ns_kernel)


def _make_mul_kernel():
    mesh = plsc.VectorSubcoreMesh(core_axis_name="c", subcore_axis_name="s")

    @functools.partial(
        pl.kernel,
        out_type=jax.ShapeDtypeStruct((_NPAD,), jnp.float32),
        mesh=mesh,
        compiler_params=_compiler_params(),
        scratch_types=[
            pltpu.VMEM((_NPW,), jnp.float32),
            pltpu.VMEM((_NPW,), jnp.float32),
        ],
    )
    def mul_kernel(h_hbm, ns_hbm, out_hbm, h_v, ns_v):
        wid = lax.axis_index("s") * 2 + lax.axis_index("c")
        base = wid * _NPW
        pltpu.sync_copy(h_hbm.at[pl.ds(base, _NPW)], h_v)
        pltpu.sync_copy(ns_hbm.at[pl.ds(base, _NPW)], ns_v)

        @pl.loop(0, _NPW // _L)
        def _(i):
            o = i * _L
            ns_v[pl.ds(o, _L)] = ns_v[pl.ds(o, _L)] * h_v[pl.ds(o, _L)]

        pltpu.sync_copy(ns_v, out_hbm.at[pl.ds(base, _NPW)])

    return mul_kernel


_mul_cache = functools.cache(_make_mul_kernel)


def _make_score_kernel():
    mesh = plsc.VectorSubcoreMesh(core_axis_name="c", subcore_axis_name="s")

    @functools.partial(
        pl.kernel,
        out_type=jax.ShapeDtypeStruct((_NB,), jnp.float32),
        mesh=mesh,
        compiler_params=_compiler_params(),
        scratch_types=[
            pltpu.VMEM((_NPAD,), jnp.float32),
            pltpu.VMEM((_BS, _BPW), jnp.int32),
            pltpu.VMEM((_BS, _BPW), jnp.float32),
            pltpu.VMEM((_BPW,), jnp.float32),
            pltpu.SemaphoreType.DMA,
        ],
    )
    def score_kernel(comb_hbm, bagsT_hbm, alphaT_hbm, out_hbm, tab_v, idx_v,
                     a_v, o_v, sem):
        wid = lax.axis_index("s") * 2 + lax.axis_index("c")
        base = wid * _BPW
        tab_cps = [
            pltpu.async_copy(comb_hbm.at[pl.ds(k * (_NPAD // 4), _NPAD // 4)],
                             tab_v.at[pl.ds(k * (_NPAD // 4), _NPAD // 4)],
                             sem)
            for k in range(4)]
        pltpu.sync_copy(bagsT_hbm.at[:, pl.ds(base, _BPW)], idx_v)
        pltpu.sync_copy(alphaT_hbm.at[:, pl.ds(base, _BPW)], a_v)
        for cp in tab_cps:
            cp.wait()

        @pl.loop(0, _BPW // _L)
        def _(i):
            o = i * _L
            acc = (plsc.load_gather(tab_v, [idx_v[0, pl.ds(o, _L)]])
                   * a_v[0, pl.ds(o, _L)])
            for j in range(1, _BS):
                acc = acc + (plsc.load_gather(tab_v, [idx_v[j, pl.ds(o, _L)]])
                             * a_v[j, pl.ds(o, _L)])
            o_v[pl.ds(o, _L)] = acc

        pltpu.sync_copy(o_v, out_hbm.at[pl.ds(base, _BPW)])

    return score_kernel


_score_cache = functools.cache(_make_score_kernel)


def kernel(x, sampled_bags, alpha_values, theta_w, node_weights, neighbors):
    h = _proj(x, theta_w)
    nbrT = neighbors.T
    tail = jnp.pad(neighbors[_TSTART:].T,
                   ((0, 0), (0, _TW - (_N - _TSTART))))
    ns = _ns_cache()(nbrT, tail, node_weights)
    comb = _mul_cache()(h, ns)
    bagsT = sampled_bags.T
    alphaT = alpha_values[:, :, 0].T
    return _score_cache()(comb, bagsT, alphaT)


# speedup vs baseline: 1.6487x; 1.0371x over previous
import dataclasses
import functools

import jax
import jax.numpy as jnp
from jax import lax
from jax.experimental import pallas as pl
from jax.experimental.pallas import tpu as pltpu
from jax.experimental.pallas import tpu_sc as plsc

_N = 100000
_D = 128
_DEG = 16
_NB = 4096
_BS = 32

_W = 32
_ABLK = 20480
_NPAD = 5 * _ABLK
_NPW = _NPAD // _W
_CHB = 640
_BPW = _NB // _W
_L = 16
_TSTART = (_N // 128) * 128
_TW = 128


def _compiler_params():
    cp = pltpu.CompilerParams()
    if "needs_layout_passes" in pltpu.CompilerParams.__dataclass_fields__:
        cp = dataclasses.replace(cp, needs_layout_passes=False)
    return cp


def _proj_body(x_ref, t_ref, h_ref):
    h = lax.dot_general(
        t_ref[...], x_ref[...], (((0,), (1,)), ((), ())),
        preferred_element_type=jnp.float32)
    h_ref[...] = h[0, :]


_proj = pl.pallas_call(
    _proj_body,
    grid=(_NPAD // _ABLK,),
    in_specs=[
        pl.BlockSpec((_ABLK, _D), lambda i: (i, 0)),
        pl.BlockSpec((_D, 1), lambda i: (0, 0)),
    ],
    out_specs=pl.BlockSpec((_ABLK,), lambda i: (i,)),
    out_shape=jax.ShapeDtypeStruct((_NPAD,), jnp.float32),
    compiler_params=pltpu.CompilerParams(
        dimension_semantics=("parallel",)),
)


def _make_ns_kernel():
    mesh = plsc.VectorSubcoreMesh(core_axis_name="c", subcore_axis_name="s")

    @functools.partial(
        pl.kernel,
        out_type=jax.ShapeDtypeStruct((_NPAD,), jnp.float32),
        mesh=mesh,
        compiler_params=_compiler_params(),
        scratch_types=[
            pltpu.VMEM((_N,), jnp.float32),
            pltpu.VMEM((_DEG, _CHB), jnp.int32),
            pltpu.VMEM((_DEG, _CHB), jnp.int32),
            pltpu.VMEM((_CHB,), jnp.float32),
            pltpu.VMEM((_DEG, _TW), jnp.int32),
            pltpu.VMEM((_TW,), jnp.float32),
            pltpu.SemaphoreType.DMA,
            pltpu.SemaphoreType.DMA,
            pltpu.SemaphoreType.DMA,
        ],
    )
    def ns_kernel(nbrT_hbm, tail_hbm, nw_hbm, out_hbm, nw_v, nbr0_v, nbr1_v,
                  o_v, t_v, to_v, sem_nw, sem0, sem1):
        wid = lax.axis_index("s") * 2 + lax.axis_index("c")
        nw_cps = []
        for st, ln in ((0, 25088), (25088, 25088), (50176, 25088),
                       (75264, _N - 75264)):
            nw_cps.append(pltpu.async_copy(
                nw_hbm.at[pl.ds(st, ln)], nw_v.at[pl.ds(st, ln)], sem_nw))
        base0 = wid * _NPW
        nchunks = _NPW // _CHB
        bufs = (nbr0_v, nbr1_v)
        sems = (sem0, sem1)

        def chunk_base(c):
            return jnp.minimum(base0 + c * _CHB, _TSTART - _CHB)

        cps = [None, None]
        cps[0] = pltpu.async_copy(
            nbrT_hbm.at[:, pl.ds(chunk_base(0), _CHB)], bufs[0], sems[0])
        for cp in nw_cps:
            cp.wait()
        for c in range(nchunks):
            if c + 1 < nchunks:
                nxt = (c + 1) % 2
                cps[nxt] = pltpu.async_copy(
                    nbrT_hbm.at[:, pl.ds(chunk_base(c + 1), _CHB)],
                    bufs[nxt], sems[nxt])
            cps[c % 2].wait()
            nbr_v = bufs[c % 2]

            @pl.loop(0, _CHB // _L)
            def _(i):
                o = i * _L
                accs = [plsc.load_gather(nw_v, [nbr_v[d, pl.ds(o, _L)]])
                        for d in range(4)]
                for d in range(4, _DEG):
                    accs[d % 4] = accs[d % 4] + plsc.load_gather(
                        nw_v, [nbr_v[d, pl.ds(o, _L)]])
                o_v[pl.ds(o, _L)] = ((accs[0] + accs[1])
                                     + (accs[2] + accs[3]))

            pltpu.sync_copy(o_v, out_hbm.at[pl.ds(chunk_base(c), _CHB)])

        @pl.when(wid == _W - 1)
        def _():
            pltpu.sync_copy(tail_hbm, t_v)

            @pl.loop(0, _TW // _L)
            def _(i):
                o = i * _L
                acc = plsc.load_gather(nw_v, [t_v[0, pl.ds(o, _L)]])
                for d in range(1, _DEG):
                    acc = acc + plsc.load_gather(nw_v,
                                                 [t_v[d, pl.ds(o, _L)]])
                to_v[pl.ds(o, _L)] = acc

            pltpu.sync_copy(to_v, out_hbm.at[pl.ds(_TSTART, _TW)])

    return ns_kernel


_ns_cache = functools.cache(_make_ns_kernel)


def _mul_body(h_ref, ns_ref, out_ref):
    out_ref[...] = h_ref[...] * ns_ref[...]


_mul = pl.pallas_call(
    _mul_body,
    out_shape=jax.ShapeDtypeStruct((_NPAD,), jnp.float32),
)


def _make_score_kernel():
    mesh = plsc.VectorSubcoreMesh(core_axis_name="c", subcore_axis_name="s")

    @functools.partial(
        pl.kernel,
        out_type=jax.ShapeDtypeStruct((_NB,), jnp.float32),
        mesh=mesh,
        compiler_params=_compiler_params(),
        scratch_types=[
            pltpu.VMEM((_NPAD,), jnp.float32),
            pltpu.VMEM((_BS, _BPW), jnp.int32),
            pltpu.VMEM((_BS, _BPW), jnp.float32),
            pltpu.VMEM((_BPW,), jnp.float32),
            pltpu.SemaphoreType.DMA,
        ],
    )
    def score_kernel(comb_hbm, bagsT_hbm, alphaT_hbm, out_hbm, tab_v, idx_v,
                     a_v, o_v, sem):
        wid = lax.axis_index("s") * 2 + lax.axis_index("c")
        base = wid * _BPW
        tab_cps = [
            pltpu.async_copy(comb_hbm.at[pl.ds(k * (_NPAD // 4), _NPAD // 4)],
                             tab_v.at[pl.ds(k * (_NPAD // 4), _NPAD // 4)],
                             sem)
            for k in range(4)]
        pltpu.sync_copy(bagsT_hbm.at[:, pl.ds(base, _BPW)], idx_v)
        pltpu.sync_copy(alphaT_hbm.at[:, pl.ds(base, _BPW)], a_v)
        for cp in tab_cps:
            cp.wait()

        @pl.loop(0, _BPW // _L)
        def _(i):
            o = i * _L
            acc = (plsc.load_gather(tab_v, [idx_v[0, pl.ds(o, _L)]])
                   * a_v[0, pl.ds(o, _L)])
            for j in range(1, _BS):
                acc = acc + (plsc.load_gather(tab_v, [idx_v[j, pl.ds(o, _L)]])
                             * a_v[j, pl.ds(o, _L)])
            o_v[pl.ds(o, _L)] = acc

        pltpu.sync_copy(o_v, out_hbm.at[pl.ds(base, _BPW)])

    return score_kernel


_score_cache = functools.cache(_make_score_kernel)


def kernel(x, sampled_bags, alpha_values, theta_w, node_weights, neighbors):
    h = _proj(x, theta_w)
    nbrT = neighbors.T
    tail = jnp.pad(neighbors[_TSTART:].T,
                   ((0, 0), (0, _TW - (_N - _TSTART))))
    ns = _ns_cache()(nbrT, tail, node_weights)
    comb = _mul(h, ns)
    bagsT = sampled_bags.T
    alphaT = alpha_values[:, :, 0].T
    return _score_cache()(comb, bagsT, alphaT)
